# baseline (speedup 1.0000x reference)
import functools

import jax
import jax.numpy as jnp
from jax import lax
from jax.experimental import pallas as pl
from jax.experimental.pallas import tpu as pltpu
from jax.experimental.pallas import tpu_sc as plsc

N = 10000
E = 320000
D = 128
H = 32
HH = H // 2

NC = 2
NS = 16
CHUNK = 128
NCH = 2560
E2 = NCH * CHUNK
CH1 = NCH // NS
NP = 10240
RPT = NP // NS

_SC_PARAMS = pltpu.CompilerParams(use_tc_tiling_on_sc=False,
                                  needs_layout_passes=False)


def _matmul1_body(x_ref, w_ref, b_ref, xacat_ref, ycat_ref):
    r = jnp.dot(x_ref[...], w_ref[...],
                preferred_element_type=jnp.float32) + b_ref[...]
    xacat_ref[0:NP, :] = r[:, 0:HH]
    xacat_ref[NP:, :] = r[:, HH:H]
    ycat_ref[0:NP, :] = r[:, H:H + HH]
    ycat_ref[NP:, :] = r[:, H + HH:]


def _layer1_matmul(xp, W1cat, b1cat):
    return pl.pallas_call(
        _matmul1_body,
        out_shape=(jax.ShapeDtypeStruct((2 * NP, HH), jnp.float32),
                   jax.ShapeDtypeStruct((2 * NP, HH), jnp.float32)),
    )(xp, W1cat, b1cat)


def _sc_agg_body(ycat_hbm, xacat_hbm, row_hbm, col01_hbm, zf_hbm, zd_hbm,
                 wv_hbm,
                 h2p_hbm, z2p_hbm, degp_hbm,
                 acc_sh, deg_sh, row_v, col_v, buf0, buf1, ones_v,
                 xa_v, acc_v, rdeg_v, dgc_v, h2_v, z2_v, wv_v,
                 gs0, gs1, ssem):
    c = lax.axis_index("c")
    s = lax.axis_index("s")

    pltpu.async_copy(zf_hbm.at[pl.ds(s * RPT, RPT)],
                     acc_sh.at[pl.ds(s * RPT, RPT)], ssem)
    pltpu.async_copy(zd_hbm.at[pl.ds(s * RPT, RPT)],
                     deg_sh.at[pl.ds(s * RPT, RPT)], ssem)
    pltpu.async_copy(row_hbm.at[pl.ds(s * CH1, CH1)], row_v, ssem)
    pltpu.async_copy(col01_hbm.at[pl.ds(c * NCH + s * CH1, CH1)], col_v, ssem)
    pltpu.async_copy(wv_hbm, wv_v, ssem)

    def init_ones(k, carry):
        ones_v[pl.ds(k * 16, 16)] = jnp.ones((16,), jnp.float32)
        return carry
    lax.fori_loop(0, CHUNK // 16, init_ones, 0)

    pltpu.make_async_copy(zf_hbm.at[pl.ds(s * RPT, RPT)],
                          acc_sh.at[pl.ds(s * RPT, RPT)], ssem).wait()
    pltpu.make_async_copy(zd_hbm.at[pl.ds(s * RPT, RPT)],
                          deg_sh.at[pl.ds(s * RPT, RPT)], ssem).wait()
    pltpu.make_async_copy(row_hbm.at[pl.ds(s * CH1, CH1)], row_v, ssem).wait()
    pltpu.make_async_copy(col01_hbm.at[pl.ds(c * NCH + s * CH1, CH1)],
                          col_v, ssem).wait()
    pltpu.make_async_copy(wv_hbm, wv_v, ssem).wait()

    plsc.subcore_barrier()

    pltpu.async_copy(ycat_hbm.at[col_v.at[0]], buf0, gs0)
    pltpu.async_copy(ycat_hbm.at[col_v.at[1]], buf1, gs1)

    def chunk(j2, carry):
        j0 = 2 * j2
        j1 = j0 + 1
        pltpu.make_async_copy(ycat_hbm.at[col_v.at[j0]], buf0, gs0).wait()
        pltpu.sync_copy(buf0, acc_sh.at[row_v.at[j0]], add=True)
        pltpu.async_copy(ycat_hbm.at[col_v.at[j0 + 2]], buf0, gs0)
        pltpu.sync_copy(ones_v, deg_sh.at[row_v.at[j0]], add=True)
        pltpu.make_async_copy(ycat_hbm.at[col_v.at[j1]], buf1, gs1).wait()
        pltpu.sync_copy(buf1, acc_sh.at[row_v.at[j1]], add=True)
        pltpu.async_copy(ycat_hbm.at[col_v.at[j1 + 2]], buf1, gs1)
        pltpu.sync_copy(ones_v, deg_sh.at[row_v.at[j1]], add=True)
        return carry
    lax.fori_loop(0, CH1 // 2 - 1, chunk, 0)

    pltpu.make_async_copy(ycat_hbm.at[col_v.at[CH1 - 2]], buf0, gs0).wait()
    pltpu.sync_copy(buf0, acc_sh.at[row_v.at[CH1 - 2]], add=True)
    pltpu.sync_copy(ones_v, deg_sh.at[row_v.at[CH1 - 2]], add=True)
    pltpu.make_async_copy(ycat_hbm.at[col_v.at[CH1 - 1]], buf1, gs1).wait()
    pltpu.sync_copy(buf1, acc_sh.at[row_v.at[CH1 - 1]], add=True)
    pltpu.sync_copy(ones_v, deg_sh.at[row_v.at[CH1 - 1]], add=True)

    plsc.subcore_barrier()

    pltpu.async_copy(xacat_hbm.at[pl.ds(c * NP + s * RPT, RPT)], xa_v, ssem)
    pltpu.sync_copy(acc_sh.at[pl.ds(s * RPT, RPT)], acc_v)
    pltpu.sync_copy(deg_sh.at[pl.ds(s * RPT, RPT)], rdeg_v)
    pltpu.make_async_copy(xacat_hbm.at[pl.ds(c * NP + s * RPT, RPT)],
                          xa_v, ssem).wait()

    def degloop(g, carry):
        dg = jnp.maximum(rdeg_v[pl.ds(g * 16, 16)], 1.0)
        dgc_v[pl.ds(g * 16, 16)] = dg
        rdeg_v[pl.ds(g * 16, 16)] = 1.0 / dg
        return carry
    lax.fori_loop(0, RPT // 16, degloop, 0)

    pltpu.sync_copy(dgc_v, degp_hbm.at[pl.ds(c * NP + s * RPT, RPT)])

    ii = lax.iota(jnp.int32, 16)
    cidx = jnp.zeros((16,), jnp.int32) + c
    cf = (1 - c).astype(jnp.float32)
    b2base = wv_v[4, pl.ds(0, 16)] * cf

    def midloop(g, carry):
        rd = rdeg_v[pl.ds(g * 16, 16)]
        idxn = ii + g * 16
        h2acc = b2base
        z2acc = jnp.zeros((16,), jnp.float32)
        for k in range(HH):
            k16 = jnp.full((16,), k, jnp.int32)
            av = plsc.load_gather(acc_v, [idxn, k16])
            xv = plsc.load_gather(xa_v, [idxn, k16])
            hv = jnp.maximum(xv + av * rd, 0.0)
            wt = plsc.load_gather(wv_v, [cidx, k16])
            wb = plsc.load_gather(wv_v, [cidx + 2, k16])
            h2acc = h2acc + hv * wt
            z2acc = z2acc + hv * wb
        h2_v[pl.ds(g * 16, 16)] = h2acc
        z2_v[pl.ds(g * 16, 16)] = z2acc
        return carry
    lax.fori_loop(0, RPT // 16, midloop, 0)

    pltpu.sync_copy(h2_v, h2p_hbm.at[pl.ds(c * NP + s * RPT, RPT)])
    pltpu.sync_copy(z2_v, z2p_hbm.at[pl.ds(c * NP + s * RPT, RPT)])


_sc_agg = functools.partial(
    pl.kernel,
    _sc_agg_body,
    out_type=(jax.ShapeDtypeStruct((2 * NP,), jnp.float32),
              jax.ShapeDtypeStruct((2 * NP,), jnp.float32),
              jax.ShapeDtypeStruct((2 * NP,), jnp.float32)),
    mesh=plsc.VectorSubcoreMesh(core_axis_name="c", subcore_axis_name="s"),
    compiler_params=_SC_PARAMS,
    scratch_types=[
        pltpu.VMEM_SHARED((NP, HH), jnp.float32),
        pltpu.VMEM_SHARED((NP,), jnp.float32),
        pltpu.VMEM((CH1, CHUNK), jnp.int32),
        pltpu.VMEM((CH1, CHUNK), jnp.int32),
        pltpu.VMEM((CHUNK, HH), jnp.float32),
        pltpu.VMEM((CHUNK, HH), jnp.float32),
        pltpu.VMEM((CHUNK,), jnp.float32),
        pltpu.VMEM((RPT, HH), jnp.float32),
        pltpu.VMEM((RPT, HH), jnp.float32),
        pltpu.VMEM((RPT,), jnp.float32),
        pltpu.VMEM((RPT,), jnp.float32),
        pltpu.VMEM((RPT,), jnp.float32),
        pltpu.VMEM((RPT,), jnp.float32),
        pltpu.VMEM((8, 16), jnp.float32),
        pltpu.SemaphoreType.DMA,
        pltpu.SemaphoreType.DMA,
        pltpu.SemaphoreType.DMA,
    ],
)()


def _sc_agg2_body(z2p_hbm, h2p_hbm, degp_hbm, row_hbm, col_hbm, zd_hbm,
                  out_hbm,
                  acc_sh, z_v, zb_v, row_v, col_v, val0, val1,
                  ha_v, hb_v, fa_v, dg_v, fin_v, ssem):
    s = lax.axis_index("s")

    pltpu.async_copy(zd_hbm.at[pl.ds(s * RPT, RPT)],
                     acc_sh.at[pl.ds(s * RPT, RPT)], ssem)
    pltpu.async_copy(z2p_hbm.at[pl.ds(0, NP)], z_v, ssem)
    pltpu.async_copy(z2p_hbm.at[pl.ds(NP, NP)], zb_v, ssem)
    pltpu.async_copy(row_hbm.at[pl.ds(s * CH1, CH1)], row_v, ssem)
    pltpu.async_copy(col_hbm.at[pl.ds(s * CH1, CH1)], col_v, ssem)
    pltpu.make_async_copy(zd_hbm.at[pl.ds(s * RPT, RPT)],
                          acc_sh.at[pl.ds(s * RPT, RPT)], ssem).wait()
    pltpu.make_async_copy(z2p_hbm.at[pl.ds(0, NP)], z_v, ssem).wait()
    pltpu.make_async_copy(z2p_hbm.at[pl.ds(NP, NP)], zb_v, ssem).wait()
    pltpu.make_async_copy(row_hbm.at[pl.ds(s * CH1, CH1)], row_v, ssem).wait()
    pltpu.make_async_copy(col_hbm.at[pl.ds(s * CH1, CH1)], col_v, ssem).wait()

    def zsum(i, carry):
        z_v[pl.ds(i * 16, 16)] = z_v[pl.ds(i * 16, 16)] + zb_v[pl.ds(i * 16, 16)]
        return carry
    lax.fori_loop(0, NP // 16, zsum, 0)

    plsc.subcore_barrier()

    def fill(j, val_v):
        for k in range(CHUNK // 16):
            cidx = col_v[j, pl.ds(k * 16, 16)]
            val_v[pl.ds(k * 16, 16)] = plsc.load_gather(z_v, [cidx])

    fill(0, val0)
    pltpu.async_copy(val0, acc_sh.at[row_v.at[0]], ssem, add=True)

    def chunk(j2, carry):
        j0 = 2 * j2
        j1 = j0 + 1
        fill(j1, val1)
        pltpu.make_async_copy(val0, acc_sh.at[row_v.at[j0]], ssem).wait()
        pltpu.async_copy(val1, acc_sh.at[row_v.at[j1]], ssem, add=True)
        fill(j0 + 2, val0)
        pltpu.make_async_copy(val1, acc_sh.at[row_v.at[j1]], ssem).wait()
        pltpu.async_copy(val0, acc_sh.at[row_v.at[j0 + 2]], ssem, add=True)
        return carry
    lax.fori_loop(0, CH1 // 2 - 1, chunk, 0)

    fill(CH1 - 1, val1)
    pltpu.make_async_copy(val0, acc_sh.at[row_v.at[CH1 - 2]], ssem).wait()
    pltpu.async_copy(val1, acc_sh.at[row_v.at[CH1 - 1]], ssem, add=True)
    pltpu.make_async_copy(val1, acc_sh.at[row_v.at[CH1 - 1]], ssem).wait()

    plsc.subcore_barrier()

    pltpu.async_copy(h2p_hbm.at[pl.ds(s * RPT, RPT)], ha_v, ssem)
    pltpu.async_copy(h2p_hbm.at[pl.ds(NP + s * RPT, RPT)], hb_v, ssem)
    pltpu.async_copy(degp_hbm.at[pl.ds(s * RPT, RPT)], dg_v, ssem)
    pltpu.sync_copy(acc_sh.at[pl.ds(s * RPT, RPT)], fa_v)
    pltpu.make_async_copy(h2p_hbm.at[pl.ds(s * RPT, RPT)], ha_v, ssem).wait()
    pltpu.make_async_copy(h2p_hbm.at[pl.ds(NP + s * RPT, RPT)], hb_v,
                          ssem).wait()
    pltpu.make_async_copy(degp_hbm.at[pl.ds(s * RPT, RPT)], dg_v, ssem).wait()

    def combine(k, carry):
        sl = pl.ds(k * 16, 16)
        fin_v[sl] = ha_v[sl] + hb_v[sl] + fa_v[sl] / dg_v[sl]
        return carry
    lax.fori_loop(0, RPT // 16, combine, 0)

    pltpu.sync_copy(fin_v, out_hbm.at[pl.ds(s * RPT, RPT)])


_sc_agg2 = functools.partial(
    pl.kernel,
    _sc_agg2_body,
    out_type=jax.ShapeDtypeStruct((NP,), jnp.float32),
    mesh=plsc.VectorSubcoreMesh(core_axis_name="c", subcore_axis_name="s",
                                num_cores=1),
    compiler_params=_SC_PARAMS,
    scratch_types=[
        pltpu.VMEM_SHARED((NP,), jnp.float32),
        pltpu.VMEM((NP,), jnp.float32),
        pltpu.VMEM((NP,), jnp.float32),
        pltpu.VMEM((CH1, CHUNK), jnp.int32),
        pltpu.VMEM((CH1, CHUNK), jnp.int32),
        pltpu.VMEM((CHUNK,), jnp.float32),
        pltpu.VMEM((CHUNK,), jnp.float32),
        pltpu.VMEM((RPT,), jnp.float32),
        pltpu.VMEM((RPT,), jnp.float32),
        pltpu.VMEM((RPT,), jnp.float32),
        pltpu.VMEM((RPT,), jnp.float32),
        pltpu.VMEM((RPT,), jnp.float32),
        pltpu.SemaphoreType.DMA,
    ],
)()


def kernel(x, edge_index, W1, b1, W2, b2):
    row = edge_index[0]
    col = edge_index[1]

    pad = E2 - E
    padr = N + (jnp.arange(pad, dtype=jnp.int32) % (NP - N))
    padc = jnp.arange(pad, dtype=jnp.int32) % (NP - N)
    rowp = jnp.concatenate([row, padr]).reshape(NCH, CHUNK)
    colp = jnp.concatenate([col, padc]).reshape(NCH, CHUNK)
    col01 = jnp.concatenate([colp, colp + NP], axis=0)

    xp = jnp.pad(x, ((0, NP - N), (0, 0)))
    W1cat = jnp.concatenate([W1[:D], W1[D:]], axis=1)
    b1cat = jnp.concatenate([b1, jnp.zeros((H,), jnp.float32)]).reshape(1, 2 * H)

    wt = W2[:H, 0]
    wb = W2[H:, 0]
    wpack = jnp.zeros((8, 16), jnp.float32)
    wpack = wpack.at[0].set(wt[:HH]).at[1].set(wt[HH:])
    wpack = wpack.at[2].set(wb[:HH]).at[3].set(wb[HH:])
    wpack = wpack.at[4].set(jnp.full((16,), b2[0]))

    zf = jnp.zeros((NP, HH), jnp.float32)
    zd = jnp.zeros((NP,), jnp.float32)

    xacat, ycat = _layer1_matmul(xp, W1cat, b1cat)
    h2p, z2p, degp = _sc_agg(ycat, xacat, rowp, col01, zf, zd, wpack)
    out = _sc_agg2(z2p, h2p, degp, rowp, colp, zd)
    return out[:N].reshape(N, 1)


# speedup vs baseline: 1.1481x; 1.1481x over previous
import functools

import jax
import jax.numpy as jnp
from jax import lax
from jax.experimental import pallas as pl
from jax.experimental.pallas import tpu as pltpu
from jax.experimental.pallas import tpu_sc as plsc

N = 10000
E = 320000
D = 128
H = 32

NC = 2
NS = 16
NW = NC * NS
CHUNK = 128
CH = 80
E2 = NW * CH * CHUNK
CH1 = E2 // (NS * CHUNK)
NP = 10240
RPT = NP // NS

_SC_PARAMS = pltpu.CompilerParams(use_tc_tiling_on_sc=False,
                                  needs_layout_passes=False)


def _matmul1_body(x_ref, w_ref, b_ref, xa_ref, y_ref):
    x = x_ref[...]
    xa_ref[...] = jnp.dot(x, w_ref[0:D, :],
                          preferred_element_type=jnp.float32) + b_ref[...]
    y_ref[...] = jnp.dot(x, w_ref[D:, :], preferred_element_type=jnp.float32)


def _layer1_matmul(xp, W1, b1):
    return pl.pallas_call(
        _matmul1_body,
        out_shape=(jax.ShapeDtypeStruct((NP, H), jnp.float32),
                   jax.ShapeDtypeStruct((NP, H), jnp.float32)),
    )(xp, W1, b1)


def _sc_agg_body(y_hbm, row_hbm, col_hbm, zf_hbm, zd_hbm,
                 parts_hbm, degp_hbm,
                 acc_sh, deg_sh, row_v, col_v, b0, b1, b2, b3, ones_v,
                 gs0, gs1, gs2, gs3, sc0, sc1, sc2, sc3,
                 ds0, ds1, ds2, ds3, ssem):
    c = lax.axis_index("c")
    s = lax.axis_index("s")
    wid = s * NC + c

    pltpu.async_copy(zf_hbm.at[pl.ds(s * RPT, RPT)],
                     acc_sh.at[pl.ds(s * RPT, RPT)], ssem)
    pltpu.async_copy(zd_hbm.at[pl.ds(s * RPT, RPT)],
                     deg_sh.at[pl.ds(s * RPT, RPT)], ssem)
    pltpu.async_copy(row_hbm.at[pl.ds(wid * CH, CH)], row_v, ssem)
    pltpu.async_copy(col_hbm.at[pl.ds(wid * CH, CH)], col_v, ssem)

    def init_ones(k, carry):
        ones_v[pl.ds(k * 16, 16)] = jnp.ones((16,), jnp.float32)
        return carry
    lax.fori_loop(0, CHUNK // 16, init_ones, 0)

    pltpu.make_async_copy(zf_hbm.at[pl.ds(s * RPT, RPT)],
                          acc_sh.at[pl.ds(s * RPT, RPT)], ssem).wait()
    pltpu.make_async_copy(zd_hbm.at[pl.ds(s * RPT, RPT)],
                          deg_sh.at[pl.ds(s * RPT, RPT)], ssem).wait()
    pltpu.make_async_copy(row_hbm.at[pl.ds(wid * CH, CH)], row_v, ssem).wait()
    pltpu.make_async_copy(col_hbm.at[pl.ds(wid * CH, CH)], col_v, ssem).wait()

    plsc.subcore_barrier()

    bufs = (b0, b1, b2, b3)
    gsems = (gs0, gs1, gs2, gs3)
    scsems = (sc0, sc1, sc2, sc3)
    dsems = (ds0, ds1, ds2, ds3)

    def g_start(j, x):
        pltpu.async_copy(y_hbm.at[col_v.at[j]], bufs[x], gsems[x])

    def g_wait(j, x):
        pltpu.make_async_copy(y_hbm.at[col_v.at[j]], bufs[x],
                              gsems[x]).wait()

    def sc_start(j, x):
        pltpu.async_copy(bufs[x], acc_sh.at[row_v.at[j]], scsems[x], add=True)
        pltpu.async_copy(ones_v, deg_sh.at[row_v.at[j]], dsems[x], add=True)

    def sc_wait(j, x):
        pltpu.make_async_copy(bufs[x], acc_sh.at[row_v.at[j]],
                              scsems[x]).wait()
        pltpu.make_async_copy(ones_v, deg_sh.at[row_v.at[j]],
                              dsems[x]).wait()

    g_start(0, 0)
    g_start(1, 1)
    g_wait(0, 0)
    sc_start(0, 0)
    g_start(2, 2)
    g_wait(1, 1)
    sc_start(1, 1)
    g_start(3, 3)

    def quad(j4, carry):
        base = 4 * j4 + 2
        for off in range(4):
            j = base + off
            x = (2 + off) % 4
            g_wait(j, x)
            sc_start(j, x)
            sc_wait(j - 2, (x + 2) % 4)
            g_start(j + 2, (x + 2) % 4)
        return carry
    lax.fori_loop(0, (CH - 4) // 4, quad, 0)

    g_wait(CH - 2, 2)
    sc_start(CH - 2, 2)
    sc_wait(CH - 4, 0)
    g_wait(CH - 1, 3)
    sc_start(CH - 1, 3)
    sc_wait(CH - 3, 1)
    sc_wait(CH - 2, 2)
    sc_wait(CH - 1, 3)

    plsc.subcore_barrier()

    pltpu.sync_copy(acc_sh.at[pl.ds(s * RPT, RPT)],
                    parts_hbm.at[pl.ds(c * NP + s * RPT, RPT)])
    pltpu.sync_copy(deg_sh.at[pl.ds(s * RPT, RPT)],
                    degp_hbm.at[pl.ds(c * NP + s * RPT, RPT)])


_sc_agg = functools.partial(
    pl.kernel,
    _sc_agg_body,
    out_type=(jax.ShapeDtypeStruct((2 * NP, H), jnp.float32),
              jax.ShapeDtypeStruct((2 * NP,), jnp.float32)),
    mesh=plsc.VectorSubcoreMesh(core_axis_name="c", subcore_axis_name="s"),
    compiler_params=_SC_PARAMS,
    scratch_types=[
        pltpu.VMEM_SHARED((NP, H), jnp.float32),
        pltpu.VMEM_SHARED((NP,), jnp.float32),
        pltpu.VMEM((CH, CHUNK), jnp.int32),
        pltpu.VMEM((CH, CHUNK), jnp.int32),
        pltpu.VMEM((CHUNK, H), jnp.float32),
        pltpu.VMEM((CHUNK, H), jnp.float32),
        pltpu.VMEM((CHUNK, H), jnp.float32),
        pltpu.VMEM((CHUNK, H), jnp.float32),
        pltpu.VMEM((CHUNK,), jnp.float32),
        pltpu.SemaphoreType.DMA,
        pltpu.SemaphoreType.DMA,
        pltpu.SemaphoreType.DMA,
        pltpu.SemaphoreType.DMA,
        pltpu.SemaphoreType.DMA,
        pltpu.SemaphoreType.DMA,
        pltpu.SemaphoreType.DMA,
        pltpu.SemaphoreType.DMA,
        pltpu.SemaphoreType.DMA,
        pltpu.SemaphoreType.DMA,
        pltpu.SemaphoreType.DMA,
        pltpu.SemaphoreType.DMA,
        pltpu.SemaphoreType.DMA,
    ],
)()


def _mid_body(xa_ref, p0_ref, p1_ref, d0_ref, d1_ref, w2_ref, b2_ref,
              h2_ref, z2_ref, deg_ref):
    deg = jnp.maximum(d0_ref[...] + d1_ref[...], 1.0)
    agg = (p0_ref[...] + p1_ref[...]) / deg
    h = jnp.maximum(xa_ref[...] + agg, 0.0)
    hb = jnp.dot(h, w2_ref[...],
                 preferred_element_type=jnp.float32) + b2_ref[...]
    h2_ref[...] = hb[:, 0:1]
    z2_ref[...] = hb[:, 1:2]
    deg_ref[...] = deg


def _mid(xa, p0, p1, d0, d1, W2cat, b2cat):
    return pl.pallas_call(
        _mid_body,
        out_shape=(jax.ShapeDtypeStruct((NP, 1), jnp.float32),
                   jax.ShapeDtypeStruct((NP, 1), jnp.float32),
                   jax.ShapeDtypeStruct((NP, 1), jnp.float32)),
    )(xa, p0, p1, d0, d1, W2cat, b2cat)


def _sc_agg2_body(z_hbm, h2_hbm, deg_hbm, row_hbm, col_hbm, zd_hbm,
                  out_hbm,
                  acc_sh, z_v, row_v, col_v, val0, val1, fin_v, ssem):
    s = lax.axis_index("s")

    pltpu.async_copy(zd_hbm.at[pl.ds(s * RPT, RPT)],
                     acc_sh.at[pl.ds(s * RPT, RPT)], ssem)
    pltpu.async_copy(z_hbm, z_v, ssem)
    pltpu.async_copy(row_hbm.at[pl.ds(s * CH1, CH1)], row_v, ssem)
    pltpu.async_copy(col_hbm.at[pl.ds(s * CH1, CH1)], col_v, ssem)
    pltpu.make_async_copy(zd_hbm.at[pl.ds(s * RPT, RPT)],
                          acc_sh.at[pl.ds(s * RPT, RPT)], ssem).wait()
    pltpu.make_async_copy(z_hbm, z_v, ssem).wait()
    pltpu.make_async_copy(row_hbm.at[pl.ds(s * CH1, CH1)], row_v, ssem).wait()
    pltpu.make_async_copy(col_hbm.at[pl.ds(s * CH1, CH1)], col_v, ssem).wait()

    plsc.subcore_barrier()

    def fill(j, val_v):
        for k in range(CHUNK // 16):
            cidx = col_v[j, pl.ds(k * 16, 16)]
            val_v[pl.ds(k * 16, 16)] = plsc.load_gather(z_v, [cidx])

    fill(0, val0)
    pltpu.async_copy(val0, acc_sh.at[row_v.at[0]], ssem, add=True)

    def chunk(j2, carry):
        j0 = 2 * j2
        j1 = j0 + 1
        fill(j1, val1)
        pltpu.make_async_copy(val0, acc_sh.at[row_v.at[j0]], ssem).wait()
        pltpu.async_copy(val1, acc_sh.at[row_v.at[j1]], ssem, add=True)
        fill(j0 + 2, val0)
        pltpu.make_async_copy(val1, acc_sh.at[row_v.at[j1]], ssem).wait()
        pltpu.async_copy(val0, acc_sh.at[row_v.at[j0 + 2]], ssem, add=True)
        return carry
    lax.fori_loop(0, CH1 // 2 - 1, chunk, 0)

    fill(CH1 - 1, val1)
    pltpu.make_async_copy(val0, acc_sh.at[row_v.at[CH1 - 2]], ssem).wait()
    pltpu.async_copy(val1, acc_sh.at[row_v.at[CH1 - 1]], ssem, add=True)
    pltpu.make_async_copy(val1, acc_sh.at[row_v.at[CH1 - 1]], ssem).wait()

    plsc.subcore_barrier()

    pltpu.sync_copy(acc_sh.at[pl.ds(s * RPT, RPT)], z_v.at[pl.ds(0, RPT)])
    pltpu.sync_copy(h2_hbm.at[pl.ds(s * RPT, RPT)], z_v.at[pl.ds(RPT, RPT)])
    pltpu.sync_copy(deg_hbm.at[pl.ds(s * RPT, RPT)],
                    z_v.at[pl.ds(2 * RPT, RPT)])

    def combine(k, carry):
        a = z_v[pl.ds(k * 16, 16)]
        h2 = z_v[pl.ds(RPT + k * 16, 16)]
        dg = z_v[pl.ds(2 * RPT + k * 16, 16)]
        fin_v[pl.ds(k * 16, 16)] = h2 + a / dg
        return carry
    lax.fori_loop(0, RPT // 16, combine, 0)

    pltpu.sync_copy(fin_v, out_hbm.at[pl.ds(s * RPT, RPT)])


_sc_agg2 = functools.partial(
    pl.kernel,
    _sc_agg2_body,
    out_type=jax.ShapeDtypeStruct((NP,), jnp.float32),
    mesh=plsc.VectorSubcoreMesh(core_axis_name="c", subcore_axis_name="s",
                                num_cores=1),
    compiler_params=_SC_PARAMS,
    scratch_types=[
        pltpu.VMEM_SHARED((NP,), jnp.float32),
        pltpu.VMEM((NP,), jnp.float32),
        pltpu.VMEM((CH1, CHUNK), jnp.int32),
        pltpu.VMEM((CH1, CHUNK), jnp.int32),
        pltpu.VMEM((CHUNK,), jnp.float32),
        pltpu.VMEM((CHUNK,), jnp.float32),
        pltpu.VMEM((RPT,), jnp.float32),
        pltpu.SemaphoreType.DMA,
    ],
)()


def kernel(x, edge_index, W1, b1, W2, b2):
    row = edge_index[0]
    col = edge_index[1]

    pad = E2 - E
    padr = N + (jnp.arange(pad, dtype=jnp.int32) % (NP - N))
    padc = jnp.arange(pad, dtype=jnp.int32) % (NP - N)
    rowp = jnp.concatenate([row, padr]).reshape(NW * CH, CHUNK)
    colp = jnp.concatenate([col, padc]).reshape(NW * CH, CHUNK)

    xp = jnp.pad(x, ((0, NP - N), (0, 0)))
    b1r = b1.reshape(1, H)
    W2cat = jnp.concatenate([W2[:H], W2[H:]], axis=1)
    b2cat = jnp.stack([b2[0], jnp.zeros((), jnp.float32)]).reshape(1, 2)

    zf = jnp.zeros((NP, H), jnp.float32)
    zd = jnp.zeros((NP,), jnp.float32)

    xa, y = _layer1_matmul(xp, W1, b1r)
    parts, degp = _sc_agg(y, rowp, colp, zf, zd)
    h2, z2, deg = _mid(xa, parts[:NP], parts[NP:], degp[:NP].reshape(NP, 1),
                       degp[NP:].reshape(NP, 1), W2cat, b2cat)

    out = _sc_agg2(z2.reshape(NP), h2.reshape(NP), deg.reshape(NP),
                   rowp, colp, zd)
    return out[:N].reshape(N, 1)
